# baseline (speedup 1.0000x reference)
import functools

import jax
import jax.numpy as jnp
from jax import lax
from jax.experimental import pallas as pl
from jax.experimental.pallas import tpu as pltpu
from jax.experimental.pallas import tpu_sc as plsc

NUM_EMB = 1_000_000
DIM = 64
BATCH = 4096 * 200
NW = 32
CH = 128
NBUF = 8
LAG = 4
B_PER_W = BATCH // NW
N_CH = B_PER_W // CH

_mesh = plsc.VectorSubcoreMesh(core_axis_name="c", subcore_axis_name="s")


@functools.partial(
    pl.kernel,
    mesh=_mesh,
    out_type=jax.ShapeDtypeStruct((BATCH, DIM), jnp.float32),
    scratch_types=(
        [pltpu.VMEM((N_CH, CH), jnp.int32)]
        + [pltpu.VMEM((CH, DIM), jnp.float32) for _ in range(NBUF)]
        + [pltpu.SemaphoreType.DMA for _ in range(2 * NBUF)]
    ),
    compiler_params=pltpu.CompilerParams(use_tc_tiling_on_sc=False),
)
def _gather_kernel(idx_hbm, w_hbm, out_hbm, idx_v, *rest):
    bufs = list(rest[:NBUF])
    gsem = list(rest[NBUF:2 * NBUF])
    ssem = list(rest[2 * NBUF:])

    wid = lax.axis_index("s") * 2 + lax.axis_index("c")
    base = wid * B_PER_W

    pltpu.sync_copy(idx_hbm.at[wid], idx_v)

    def start_gather(b, j):
        pltpu.async_copy(w_hbm.at[idx_v.at[j]], bufs[b], gsem[b])

    def wait_gather(b):
        pltpu.make_async_copy(w_hbm.at[idx_v.at[0]], bufs[b], gsem[b]).wait()

    def start_scatter(b, j):
        pltpu.async_copy(bufs[b], out_hbm.at[pl.ds(base + j * CH, CH)], ssem[b])

    def wait_scatter(b):
        pltpu.make_async_copy(
            bufs[b], out_hbm.at[pl.ds(base, CH)], ssem[b]
        ).wait()


    for j in range(NBUF):
        start_gather(j % NBUF, j)
        if j >= LAG:
            b2 = (j - LAG) % NBUF
            wait_gather(b2)
            start_scatter(b2, j - LAG)

    def group_body(g, carry):
        j0 = (g + 1) * NBUF
        for k in range(NBUF):
            j = j0 + k
            wait_scatter(k)
            start_gather(k, j)
            b2 = (k - LAG) % NBUF
            wait_gather(b2)
            start_scatter(b2, j - LAG)
        return carry

    lax.fori_loop(0, N_CH // NBUF - 1, group_body, 0, unroll=False)

    for j in range(N_CH, N_CH + LAG):
        b2 = (j - LAG) % NBUF
        wait_gather(b2)
        start_scatter(b2, j - LAG)
    for b in range(NBUF):
        wait_scatter(b)


def kernel(token_ids, W):
    idx = token_ids.astype(jnp.int32).reshape(NW, N_CH, CH)
    out = _gather_kernel(idx, W)
    return out.reshape(token_ids.shape[0], token_ids.shape[1], DIM)


# speedup vs baseline: 1.2209x; 1.2209x over previous
import functools

import jax
import jax.numpy as jnp
from jax import lax
from jax.experimental import pallas as pl
from jax.experimental.pallas import tpu as pltpu
from jax.experimental.pallas import tpu_sc as plsc

NUM_EMB = 1_000_000
DIM = 64
PDIM = 128
BATCH = 4096 * 200
NW = 32
CH = 128
NBUF = 4
LAG = 2
B_PER_W = BATCH // NW
N_CH = B_PER_W // CH

_mesh = plsc.VectorSubcoreMesh(core_axis_name="c", subcore_axis_name="s")


@functools.partial(
    pl.kernel,
    mesh=_mesh,
    out_type=jax.ShapeDtypeStruct((BATCH, PDIM), jnp.float32),
    scratch_types=(
        [pltpu.VMEM((N_CH, CH), jnp.int32)]
        + [pltpu.VMEM((CH, PDIM), jnp.float32) for _ in range(NBUF)]
        + [pltpu.SemaphoreType.DMA for _ in range(2 * NBUF)]
    ),
)
def _gather_kernel(idx_hbm, w_hbm, out_hbm, idx_v, *rest):
    bufs = list(rest[:NBUF])
    gsem = list(rest[NBUF:2 * NBUF])
    ssem = list(rest[2 * NBUF:])

    wid = lax.axis_index("s") * 2 + lax.axis_index("c")
    base = wid * B_PER_W

    pltpu.sync_copy(idx_hbm.at[wid], idx_v)

    def start_gather(b, j):
        pltpu.async_copy(w_hbm.at[idx_v.at[j]], bufs[b], gsem[b])

    def wait_gather(b):
        pltpu.make_async_copy(w_hbm.at[idx_v.at[0]], bufs[b], gsem[b]).wait()

    def start_scatter(b, j):
        pltpu.async_copy(bufs[b], out_hbm.at[pl.ds(base + j * CH, CH)], ssem[b])

    def wait_scatter(b):
        pltpu.make_async_copy(
            bufs[b], out_hbm.at[pl.ds(base, CH)], ssem[b]
        ).wait()


    for j in range(NBUF):
        start_gather(j % NBUF, j)
        if j >= LAG:
            b2 = (j - LAG) % NBUF
            wait_gather(b2)
            start_scatter(b2, j - LAG)

    def group_body(g, carry):
        j0 = (g + 1) * NBUF
        for k in range(NBUF):
            j = j0 + k
            wait_scatter(k)
            start_gather(k, j)
            b2 = (k - LAG) % NBUF
            wait_gather(b2)
            start_scatter(b2, j - LAG)
        return carry

    lax.fori_loop(0, N_CH // NBUF - 1, group_body, 0, unroll=False)

    for j in range(N_CH, N_CH + LAG):
        b2 = (j - LAG) % NBUF
        wait_gather(b2)
        start_scatter(b2, j - LAG)
    for b in range(NBUF):
        wait_scatter(b)


def kernel(token_ids, W):
    idx = token_ids.astype(jnp.int32).reshape(NW, N_CH, CH)
    w_pad = jnp.pad(W, ((0, 0), (0, PDIM - DIM)))
    out = _gather_kernel(idx, w_pad)
    out = out.reshape(token_ids.shape[0], token_ids.shape[1], PDIM)
    return out[:, :, :DIM]
